# baseline (speedup 1.0000x reference)
import jax
import jax.numpy as jnp
from jax import lax
from jax.experimental import pallas as pl
from jax.experimental.pallas import tpu as pltpu
from jax.experimental.pallas import tpu_sc as plsc

N_NODES = 50000
N_EDGES = 800000
N_GRAPHS = 128
NODE_EMB = 128
N_RBF = 32
EDGE_EMB = 32
CUTOFF = 6.0
SIGMA = CUTOFF / N_RBF
INV2S2 = 1.0 / (2.0 * SIGMA * SIGMA)
MU_STEP = CUTOFF / (N_RBF - 1)

NC, NS = 2, 16
NW = NC * NS
EPW = N_EDGES // NW
ECHUNK = 1000
NECHUNK = EPW // ECHUNK
NODE_WORKERS = 25
NPW = N_NODES // NODE_WORKERS


def _sc_body(cartx, carty, cartz, senders, receivers, jim3, gidx, lats_flat,
             species, table, vx_out, vy_out, vz_out, vi_out, nemb_out,
             sidx, ridx, sxb, syb, szb, rxb, ryb, rzb,
             jb3, gb, latb, vxb, vyb, vzb, vb, spid, nrows,
             cxsh, cysh, czsh, sem_in, sem_g, sem_n):
    sid = lax.axis_index("s")
    wid = sid * NC + lax.axis_index("c")

    pltpu.sync_copy(lats_flat, latb)

    @pl.when(sid == 0)
    def _():
        pltpu.sync_copy(cartx, cxsh)
        pltpu.sync_copy(carty, cysh)
        pltpu.sync_copy(cartz, czsh)
    plsc.subcore_barrier()

    iot = lax.iota(jnp.int32, 16)

    def edge_group(o):
        sx = sxb[pl.ds(o, 16)]
        sy = syb[pl.ds(o, 16)]
        sz = szb[pl.ds(o, 16)]
        rx = rxb[pl.ds(o, 16)]
        ry = ryb[pl.ds(o, 16)]
        rz = rzb[pl.ds(o, 16)]
        g9 = gb[pl.ds(o, 16)] * 9
        r3 = (o + iot) * 3
        ja = plsc.load_gather(jb3, [r3]).astype(jnp.float32)
        jb_ = plsc.load_gather(jb3, [r3 + 1]).astype(jnp.float32)
        jc = plsc.load_gather(jb3, [r3 + 2]).astype(jnp.float32)
        l00 = plsc.load_gather(latb, [g9])
        l01 = plsc.load_gather(latb, [g9 + 1])
        l02 = plsc.load_gather(latb, [g9 + 2])
        l10 = plsc.load_gather(latb, [g9 + 3])
        l11 = plsc.load_gather(latb, [g9 + 4])
        l12 = plsc.load_gather(latb, [g9 + 5])
        l20 = plsc.load_gather(latb, [g9 + 6])
        l21 = plsc.load_gather(latb, [g9 + 7])
        l22 = plsc.load_gather(latb, [g9 + 8])
        vx = rx + (l00 * ja + l10 * jb_ + l20 * jc) - sx
        vy = ry + (l01 * ja + l11 * jb_ + l21 * jc) - sy
        vz = rz + (l02 * ja + l12 * jb_ + l22 * jc) - sz
        vxb[pl.ds(o, 16)] = vx
        vyb[pl.ds(o, 16)] = vy
        vzb[pl.ds(o, 16)] = vz
        plsc.store_scatter(vb, [r3], vx)
        plsc.store_scatter(vb, [r3 + 1], vy)
        plsc.store_scatter(vb, [r3 + 2], vz)

    def chunk_body(ci, carry):
        base = wid * EPW + ci * ECHUNK
        sl_e = pl.ds(base, ECHUNK)
        cps = [pltpu.async_copy(senders.at[sl_e], sidx, sem_in),
               pltpu.async_copy(receivers.at[sl_e], ridx, sem_in),
               pltpu.async_copy(gidx.at[sl_e], gb, sem_in),
               pltpu.async_copy(jim3.at[pl.ds(3 * base, 3 * ECHUNK)], jb3,
                                sem_in)]
        for cp in cps:
            cp.wait()
        gcps = []
        for j in range(8):
            n = 128 if j < 7 else ECHUNK - 7 * 128
            sl = pl.ds(j * 128, n)
            for tab, idx, dst in ((cxsh, sidx, sxb), (cysh, sidx, syb),
                                  (czsh, sidx, szb), (cxsh, ridx, rxb),
                                  (cysh, ridx, ryb), (czsh, ridx, rzb)):
                gcps.append(pltpu.async_copy(tab.at[idx.at[sl]], dst.at[sl],
                                             sem_g))
        for cp in gcps:
            cp.wait()

        def g_body(i, c):
            edge_group(i * 16)
            return c
        lax.fori_loop(0, ECHUNK // 16, g_body, 0)
        edge_group(ECHUNK - 16)
        ocps = [pltpu.async_copy(vxb, vx_out.at[sl_e], sem_in),
                pltpu.async_copy(vyb, vy_out.at[sl_e], sem_in),
                pltpu.async_copy(vzb, vz_out.at[sl_e], sem_in),
                pltpu.async_copy(vb, vi_out.at[pl.ds(3 * base, 3 * ECHUNK)],
                                 sem_in)]
        for cp in ocps:
            cp.wait()
        return carry

    lax.fori_loop(0, NECHUNK, chunk_body, 0)

    @pl.when(wid < NODE_WORKERS)
    def _():
        nb = wid * NPW
        for j in range(16):
            n = 128 if j < 15 else NPW - 15 * 128
            sl = pl.ds(0, n)
            pltpu.sync_copy(species.at[pl.ds(nb + j * 128, n)], spid.at[sl])
            pltpu.async_copy(table.at[spid.at[sl]], nrows.at[sl],
                             sem_n).wait()
            pltpu.sync_copy(nrows.at[sl],
                            nemb_out.at[pl.ds(nb + j * 128, n)])


_sc_call = pl.kernel(
    _sc_body,
    out_type=[
        jax.ShapeDtypeStruct((N_EDGES,), jnp.float32),
        jax.ShapeDtypeStruct((N_EDGES,), jnp.float32),
        jax.ShapeDtypeStruct((N_EDGES,), jnp.float32),
        jax.ShapeDtypeStruct((3 * N_EDGES,), jnp.float32),
        jax.ShapeDtypeStruct((N_NODES, NODE_EMB), jnp.float32),
    ],
    mesh=plsc.VectorSubcoreMesh(core_axis_name="c", subcore_axis_name="s"),
    compiler_params=pltpu.CompilerParams(needs_layout_passes=False,
                                         use_tc_tiling_on_sc=False),
    scratch_types=[
        pltpu.VMEM((ECHUNK,), jnp.int32),
        pltpu.VMEM((ECHUNK,), jnp.int32),
        pltpu.VMEM((ECHUNK,), jnp.float32),
        pltpu.VMEM((ECHUNK,), jnp.float32),
        pltpu.VMEM((ECHUNK,), jnp.float32),
        pltpu.VMEM((ECHUNK,), jnp.float32),
        pltpu.VMEM((ECHUNK,), jnp.float32),
        pltpu.VMEM((ECHUNK,), jnp.float32),
        pltpu.VMEM((3 * ECHUNK,), jnp.int32),
        pltpu.VMEM((ECHUNK,), jnp.int32),
        pltpu.VMEM((N_GRAPHS * 9,), jnp.float32),
        pltpu.VMEM((ECHUNK,), jnp.float32),
        pltpu.VMEM((ECHUNK,), jnp.float32),
        pltpu.VMEM((ECHUNK,), jnp.float32),
        pltpu.VMEM((3 * ECHUNK,), jnp.float32),
        pltpu.VMEM((128,), jnp.int32),
        pltpu.VMEM((128, NODE_EMB), jnp.float32),
        pltpu.VMEM_SHARED((N_NODES,), jnp.float32),
        pltpu.VMEM_SHARED((N_NODES,), jnp.float32),
        pltpu.VMEM_SHARED((N_NODES,), jnp.float32),
        pltpu.SemaphoreType.DMA,
        pltpu.SemaphoreType.DMA,
        pltpu.SemaphoreType.DMA,
    ],
)

TBR = 64
TBE = TBR * 128
NROW = N_EDGES // 128
TGRID = (NROW + TBR - 1) // TBR


def _tc_body(vx_ref, vy_ref, vz_ref, w_ref, b_ref, dist_ref, emb_ref, rbf_t):
    x = vx_ref[...]
    y = vy_ref[...]
    z = vz_ref[...]
    d = jnp.sqrt(x * x + y * y + z * z + 1e-12)
    dist_ref[...] = d
    mu = lax.broadcasted_iota(jnp.int32, (N_RBF, 128), 0).astype(
        jnp.float32) * MU_STEP
    for r in range(TBR):
        db = jnp.broadcast_to(d[r:r + 1, :], (N_RBF, 128))
        delta = db - mu
        rbf_t[:, r * 128:(r + 1) * 128] = jnp.exp(-(delta * delta) * INV2S2)
    emb = lax.dot_general(rbf_t[...], w_ref[...], (((0,), (0,)), ((), ())),
                          preferred_element_type=jnp.float32)
    emb_ref[...] = emb + b_ref[...]


_tc_call = pl.pallas_call(
    _tc_body,
    grid=(TGRID,),
    in_specs=[
        pl.BlockSpec((TBR, 128), lambda i: (i, 0)),
        pl.BlockSpec((TBR, 128), lambda i: (i, 0)),
        pl.BlockSpec((TBR, 128), lambda i: (i, 0)),
        pl.BlockSpec((N_RBF, EDGE_EMB), lambda i: (0, 0)),
        pl.BlockSpec((1, EDGE_EMB), lambda i: (0, 0)),
    ],
    out_specs=[
        pl.BlockSpec((TBR, 128), lambda i: (i, 0)),
        pl.BlockSpec((TBE, EDGE_EMB), lambda i: (i, 0)),
    ],
    out_shape=[
        jax.ShapeDtypeStruct((NROW, 128), jnp.float32),
        jax.ShapeDtypeStruct((N_EDGES, EDGE_EMB), jnp.float32),
    ],
    scratch_shapes=[pltpu.VMEM((N_RBF, TBE), jnp.float32)],
)


def kernel(cart, lats, senders, receivers, to_jimage, edge_graph_i, species,
           W_proj, b_proj, species_table):
    lats_flat = lax.reduce_precision(lats, exponent_bits=8,
                                     mantissa_bits=7).reshape(-1)
    cartx, carty, cartz = cart[:, 0], cart[:, 1], cart[:, 2]
    vx, vy, vz, vflat, node_emb = _sc_call(cartx, carty, cartz, senders,
                                           receivers, to_jimage.reshape(-1),
                                           edge_graph_i, lats_flat, species,
                                           species_table)
    dist2d, edge_emb = _tc_call(vx.reshape(NROW, 128), vy.reshape(NROW, 128),
                                vz.reshape(NROW, 128), W_proj,
                                b_proj.reshape(1, EDGE_EMB))
    vecs = vflat.reshape(N_EDGES, 3)
    return node_emb, edge_emb, vecs, dist2d.reshape(N_EDGES)


# speedup vs baseline: 3.3754x; 3.3754x over previous
import jax
import jax.numpy as jnp
from jax import lax
from jax.experimental import pallas as pl
from jax.experimental.pallas import tpu as pltpu
from jax.experimental.pallas import tpu_sc as plsc

N_NODES = 50000
N_EDGES = 800000
N_GRAPHS = 128
NODE_EMB = 128
N_RBF = 32
EDGE_EMB = 32
CUTOFF = 6.0
SIGMA = CUTOFF / N_RBF
INV2S2 = 1.0 / (2.0 * SIGMA * SIGMA)
MU_STEP = CUTOFF / (N_RBF - 1)

NC, NS = 2, 16
NW = NC * NS
EPW = N_EDGES // NW
ECHUNK = 1000
NECHUNK = EPW // ECHUNK
NODE_WORKERS = 25
NPW = N_NODES // NODE_WORKERS


def _sc_body(cartx, carty, cartz, senders, receivers, jx, jy, jz, gidx,
             lats_flat, species, table,
             vx_out, vy_out, vz_out, vi_out, nemb_out,
             sidx, ridx, sxb, syb, szb, rxb, ryb, rzb,
             jxb, jyb, jzb, gb, latb, vxb, vyb, vzb, vb, spid, nrows,
             cxsh, cysh, czsh, sem_in, sem_g, sem_n):
    sid = lax.axis_index("s")
    wid = sid * NC + lax.axis_index("c")

    pltpu.sync_copy(lats_flat, latb)

    @pl.when(sid == 0)
    def _():
        pltpu.sync_copy(cartx, cxsh)
        pltpu.sync_copy(carty, cysh)
        pltpu.sync_copy(cartz, czsh)
    plsc.subcore_barrier()

    iot = lax.iota(jnp.int32, 16)

    def edge_group(o):
        sx = sxb[pl.ds(o, 16)]
        sy = syb[pl.ds(o, 16)]
        sz = szb[pl.ds(o, 16)]
        rx = rxb[pl.ds(o, 16)]
        ry = ryb[pl.ds(o, 16)]
        rz = rzb[pl.ds(o, 16)]
        g9 = gb[pl.ds(o, 16)] * 9
        r3 = (o + iot) * 3
        ja = jxb[pl.ds(o, 16)]
        jb_ = jyb[pl.ds(o, 16)]
        jc = jzb[pl.ds(o, 16)]
        l00 = plsc.load_gather(latb, [g9])
        l01 = plsc.load_gather(latb, [g9 + 1])
        l02 = plsc.load_gather(latb, [g9 + 2])
        l10 = plsc.load_gather(latb, [g9 + 3])
        l11 = plsc.load_gather(latb, [g9 + 4])
        l12 = plsc.load_gather(latb, [g9 + 5])
        l20 = plsc.load_gather(latb, [g9 + 6])
        l21 = plsc.load_gather(latb, [g9 + 7])
        l22 = plsc.load_gather(latb, [g9 + 8])
        vx = rx + (l00 * ja + l10 * jb_ + l20 * jc) - sx
        vy = ry + (l01 * ja + l11 * jb_ + l21 * jc) - sy
        vz = rz + (l02 * ja + l12 * jb_ + l22 * jc) - sz
        vxb[pl.ds(o, 16)] = vx
        vyb[pl.ds(o, 16)] = vy
        vzb[pl.ds(o, 16)] = vz
        plsc.store_scatter(vb, [r3], vx)
        plsc.store_scatter(vb, [r3 + 1], vy)
        plsc.store_scatter(vb, [r3 + 2], vz)

    def chunk_body(ci, carry):
        base = wid * EPW + ci * ECHUNK
        sl_e = pl.ds(base, ECHUNK)
        cps = [pltpu.async_copy(senders.at[sl_e], sidx, sem_in),
               pltpu.async_copy(receivers.at[sl_e], ridx, sem_in),
               pltpu.async_copy(gidx.at[sl_e], gb, sem_in),
               pltpu.async_copy(jx.at[sl_e], jxb, sem_in),
               pltpu.async_copy(jy.at[sl_e], jyb, sem_in),
               pltpu.async_copy(jz.at[sl_e], jzb, sem_in)]
        for cp in cps:
            cp.wait()
        gcps = []
        for j in range(8):
            n = 128 if j < 7 else ECHUNK - 7 * 128
            sl = pl.ds(j * 128, n)
            for tab, idx, dst in ((cxsh, sidx, sxb), (cysh, sidx, syb),
                                  (czsh, sidx, szb), (cxsh, ridx, rxb),
                                  (cysh, ridx, ryb), (czsh, ridx, rzb)):
                gcps.append(pltpu.async_copy(tab.at[idx.at[sl]], dst.at[sl],
                                             sem_g))
        for cp in gcps:
            cp.wait()

        def g_body(i, c):
            edge_group(i * 16)
            return c
        lax.fori_loop(0, ECHUNK // 16, g_body, 0)
        edge_group(ECHUNK - 16)
        ocps = [pltpu.async_copy(vxb, vx_out.at[sl_e], sem_in),
                pltpu.async_copy(vyb, vy_out.at[sl_e], sem_in),
                pltpu.async_copy(vzb, vz_out.at[sl_e], sem_in),
                pltpu.async_copy(vb, vi_out.at[pl.ds(3 * base, 3 * ECHUNK)],
                                 sem_in)]
        for cp in ocps:
            cp.wait()
        return carry

    lax.fori_loop(0, NECHUNK, chunk_body, 0)

    @pl.when(wid < NODE_WORKERS)
    def _():
        nb = wid * NPW
        for j in range(16):
            n = 128 if j < 15 else NPW - 15 * 128
            sl = pl.ds(0, n)
            pltpu.sync_copy(species.at[pl.ds(nb + j * 128, n)], spid.at[sl])
            pltpu.async_copy(table.at[spid.at[sl]], nrows.at[sl],
                             sem_n).wait()
            pltpu.sync_copy(nrows.at[sl],
                            nemb_out.at[pl.ds(nb + j * 128, n)])


_sc_call = pl.kernel(
    _sc_body,
    out_type=[
        jax.ShapeDtypeStruct((N_EDGES,), jnp.float32),
        jax.ShapeDtypeStruct((N_EDGES,), jnp.float32),
        jax.ShapeDtypeStruct((N_EDGES,), jnp.float32),
        jax.ShapeDtypeStruct((3 * N_EDGES,), jnp.float32),
        jax.ShapeDtypeStruct((N_NODES, NODE_EMB), jnp.float32),
    ],
    mesh=plsc.VectorSubcoreMesh(core_axis_name="c", subcore_axis_name="s"),
    compiler_params=pltpu.CompilerParams(needs_layout_passes=False,
                                         use_tc_tiling_on_sc=False),
    scratch_types=[
        pltpu.VMEM((ECHUNK,), jnp.int32),
        pltpu.VMEM((ECHUNK,), jnp.int32),
        pltpu.VMEM((ECHUNK,), jnp.float32),
        pltpu.VMEM((ECHUNK,), jnp.float32),
        pltpu.VMEM((ECHUNK,), jnp.float32),
        pltpu.VMEM((ECHUNK,), jnp.float32),
        pltpu.VMEM((ECHUNK,), jnp.float32),
        pltpu.VMEM((ECHUNK,), jnp.float32),
        pltpu.VMEM((ECHUNK,), jnp.float32),
        pltpu.VMEM((ECHUNK,), jnp.float32),
        pltpu.VMEM((ECHUNK,), jnp.float32),
        pltpu.VMEM((ECHUNK,), jnp.int32),
        pltpu.VMEM((N_GRAPHS * 9,), jnp.float32),
        pltpu.VMEM((ECHUNK,), jnp.float32),
        pltpu.VMEM((ECHUNK,), jnp.float32),
        pltpu.VMEM((ECHUNK,), jnp.float32),
        pltpu.VMEM((3 * ECHUNK,), jnp.float32),
        pltpu.VMEM((128,), jnp.int32),
        pltpu.VMEM((128, NODE_EMB), jnp.float32),
        pltpu.VMEM_SHARED((N_NODES,), jnp.float32),
        pltpu.VMEM_SHARED((N_NODES,), jnp.float32),
        pltpu.VMEM_SHARED((N_NODES,), jnp.float32),
        pltpu.SemaphoreType.DMA,
        pltpu.SemaphoreType.DMA,
        pltpu.SemaphoreType.DMA,
    ],
)

TBR = 64
TBE = TBR * 128
NROW = N_EDGES // 128
TGRID = (NROW + TBR - 1) // TBR


def _tc_body(vx_ref, vy_ref, vz_ref, w_ref, b_ref, dist_ref, emb_ref, rbf_t):
    x = vx_ref[...]
    y = vy_ref[...]
    z = vz_ref[...]
    d = jnp.sqrt(x * x + y * y + z * z + 1e-12)
    dist_ref[...] = d
    mu = lax.broadcasted_iota(jnp.int32, (N_RBF, 128), 0).astype(
        jnp.float32) * MU_STEP
    for r in range(TBR):
        db = jnp.broadcast_to(d[r:r + 1, :], (N_RBF, 128))
        delta = db - mu
        rbf_t[:, r * 128:(r + 1) * 128] = jnp.exp(-(delta * delta) * INV2S2)
    emb = lax.dot_general(rbf_t[...], w_ref[...], (((0,), (0,)), ((), ())),
                          preferred_element_type=jnp.float32)
    emb_ref[...] = emb + b_ref[...]


_tc_call = pl.pallas_call(
    _tc_body,
    grid=(TGRID,),
    in_specs=[
        pl.BlockSpec((TBR, 128), lambda i: (i, 0)),
        pl.BlockSpec((TBR, 128), lambda i: (i, 0)),
        pl.BlockSpec((TBR, 128), lambda i: (i, 0)),
        pl.BlockSpec((N_RBF, EDGE_EMB), lambda i: (0, 0)),
        pl.BlockSpec((1, EDGE_EMB), lambda i: (0, 0)),
    ],
    out_specs=[
        pl.BlockSpec((TBR, 128), lambda i: (i, 0)),
        pl.BlockSpec((TBE, EDGE_EMB), lambda i: (i, 0)),
    ],
    out_shape=[
        jax.ShapeDtypeStruct((NROW, 128), jnp.float32),
        jax.ShapeDtypeStruct((N_EDGES, EDGE_EMB), jnp.float32),
    ],
    scratch_shapes=[pltpu.VMEM((N_RBF, TBE), jnp.float32)],
)


def kernel(cart, lats, senders, receivers, to_jimage, edge_graph_i, species,
           W_proj, b_proj, species_table):
    lats_flat = lax.reduce_precision(lats, exponent_bits=8,
                                     mantissa_bits=7).reshape(-1)
    cartx, carty, cartz = cart[:, 0], cart[:, 1], cart[:, 2]
    jimf = to_jimage.astype(jnp.float32)
    vx, vy, vz, vflat, node_emb = _sc_call(cartx, carty, cartz, senders,
                                           receivers, jimf[:, 0], jimf[:, 1],
                                           jimf[:, 2], edge_graph_i,
                                           lats_flat, species, species_table)
    dist2d, edge_emb = _tc_call(vx.reshape(NROW, 128), vy.reshape(NROW, 128),
                                vz.reshape(NROW, 128), W_proj,
                                b_proj.reshape(1, EDGE_EMB))
    vecs = vflat.reshape(N_EDGES, 3)
    return node_emb, edge_emb, vecs, dist2d.reshape(N_EDGES)


# speedup vs baseline: 5.2524x; 1.5561x over previous
import jax
import jax.numpy as jnp
from jax import lax
from jax.experimental import pallas as pl
from jax.experimental.pallas import tpu as pltpu
from jax.experimental.pallas import tpu_sc as plsc

N_NODES = 50000
N_EDGES = 800000
N_GRAPHS = 128
NODE_EMB = 128
N_RBF = 32
EDGE_EMB = 32
CUTOFF = 6.0
SIGMA = CUTOFF / N_RBF
INV2S2 = 1.0 / (2.0 * SIGMA * SIGMA)
MU_STEP = CUTOFF / (N_RBF - 1)

NC, NS = 2, 16
NW = NC * NS
EPW = N_EDGES // NW
ECHUNK = 1000
NECHUNK = EPW // ECHUNK
NODE_WORKERS = 25
NPW = N_NODES // NODE_WORKERS


def _sc_body(cartx, carty, cartz, senders, receivers, jx, jy, jz, gidx,
             lats_flat, species, table,
             vx_out, vy_out, vz_out, nemb_out,
             sidx, ridx, sxb, syb, szb, rxb, ryb, rzb,
             jxb, jyb, jzb, gb, latb, vxb, vyb, vzb, spid, nrows,
             cxsh, cysh, czsh, sem_in, sem_g, sem_n):
    sid = lax.axis_index("s")
    wid = sid * NC + lax.axis_index("c")

    pltpu.sync_copy(lats_flat, latb)

    @pl.when(sid == 0)
    def _():
        pltpu.sync_copy(cartx, cxsh)
        pltpu.sync_copy(carty, cysh)
        pltpu.sync_copy(cartz, czsh)
    plsc.subcore_barrier()

    iot = lax.iota(jnp.int32, 16)

    def edge_group(o):
        sx = sxb[pl.ds(o, 16)]
        sy = syb[pl.ds(o, 16)]
        sz = szb[pl.ds(o, 16)]
        rx = rxb[pl.ds(o, 16)]
        ry = ryb[pl.ds(o, 16)]
        rz = rzb[pl.ds(o, 16)]
        g9 = gb[pl.ds(o, 16)] * 9
        ja = jxb[pl.ds(o, 16)].astype(jnp.float32)
        jb_ = jyb[pl.ds(o, 16)].astype(jnp.float32)
        jc = jzb[pl.ds(o, 16)].astype(jnp.float32)
        l00 = plsc.load_gather(latb, [g9])
        l01 = plsc.load_gather(latb, [g9 + 1])
        l02 = plsc.load_gather(latb, [g9 + 2])
        l10 = plsc.load_gather(latb, [g9 + 3])
        l11 = plsc.load_gather(latb, [g9 + 4])
        l12 = plsc.load_gather(latb, [g9 + 5])
        l20 = plsc.load_gather(latb, [g9 + 6])
        l21 = plsc.load_gather(latb, [g9 + 7])
        l22 = plsc.load_gather(latb, [g9 + 8])
        vxb[pl.ds(o, 16)] = rx + (l00 * ja + l10 * jb_ + l20 * jc) - sx
        vyb[pl.ds(o, 16)] = ry + (l01 * ja + l11 * jb_ + l21 * jc) - sy
        vzb[pl.ds(o, 16)] = rz + (l02 * ja + l12 * jb_ + l22 * jc) - sz

    def chunk_body(ci, carry):
        base = wid * EPW + ci * ECHUNK
        sl_e = pl.ds(base, ECHUNK)
        cps = [pltpu.async_copy(senders.at[sl_e], sidx, sem_in),
               pltpu.async_copy(receivers.at[sl_e], ridx, sem_in),
               pltpu.async_copy(gidx.at[sl_e], gb, sem_in),
               pltpu.async_copy(jx.at[sl_e], jxb, sem_in),
               pltpu.async_copy(jy.at[sl_e], jyb, sem_in),
               pltpu.async_copy(jz.at[sl_e], jzb, sem_in)]
        for cp in cps:
            cp.wait()
        gcps = []
        for j in range(8):
            n = 128 if j < 7 else ECHUNK - 7 * 128
            sl = pl.ds(j * 128, n)
            for tab, idx, dst in ((cxsh, sidx, sxb), (cysh, sidx, syb),
                                  (czsh, sidx, szb), (cxsh, ridx, rxb),
                                  (cysh, ridx, ryb), (czsh, ridx, rzb)):
                gcps.append(pltpu.async_copy(tab.at[idx.at[sl]], dst.at[sl],
                                             sem_g))
        for cp in gcps:
            cp.wait()

        def g_body(i, c):
            edge_group(i * 16)
            return c
        lax.fori_loop(0, ECHUNK // 16, g_body, 0)
        edge_group(ECHUNK - 16)
        ocps = [pltpu.async_copy(vxb, vx_out.at[sl_e], sem_in),
                pltpu.async_copy(vyb, vy_out.at[sl_e], sem_in),
                pltpu.async_copy(vzb, vz_out.at[sl_e], sem_in)]
        for cp in ocps:
            cp.wait()
        return carry

    lax.fori_loop(0, NECHUNK, chunk_body, 0)

    @pl.when(wid < NODE_WORKERS)
    def _():
        nb = wid * NPW
        for j in range(16):
            n = 128 if j < 15 else NPW - 15 * 128
            sl = pl.ds(0, n)
            pltpu.sync_copy(species.at[pl.ds(nb + j * 128, n)], spid.at[sl])
            pltpu.async_copy(table.at[spid.at[sl]], nrows.at[sl],
                             sem_n).wait()
            pltpu.sync_copy(nrows.at[sl],
                            nemb_out.at[pl.ds(nb + j * 128, n)])


_sc_call = pl.kernel(
    _sc_body,
    out_type=[
        jax.ShapeDtypeStruct((N_EDGES,), jnp.float32),
        jax.ShapeDtypeStruct((N_EDGES,), jnp.float32),
        jax.ShapeDtypeStruct((N_EDGES,), jnp.float32),
        jax.ShapeDtypeStruct((N_NODES, NODE_EMB), jnp.float32),
    ],
    mesh=plsc.VectorSubcoreMesh(core_axis_name="c", subcore_axis_name="s"),
    compiler_params=pltpu.CompilerParams(needs_layout_passes=False,
                                         use_tc_tiling_on_sc=False),
    scratch_types=[
        pltpu.VMEM((ECHUNK,), jnp.int32),
        pltpu.VMEM((ECHUNK,), jnp.int32),
        pltpu.VMEM((ECHUNK,), jnp.float32),
        pltpu.VMEM((ECHUNK,), jnp.float32),
        pltpu.VMEM((ECHUNK,), jnp.float32),
        pltpu.VMEM((ECHUNK,), jnp.float32),
        pltpu.VMEM((ECHUNK,), jnp.float32),
        pltpu.VMEM((ECHUNK,), jnp.float32),
        pltpu.VMEM((ECHUNK,), jnp.int32),
        pltpu.VMEM((ECHUNK,), jnp.int32),
        pltpu.VMEM((ECHUNK,), jnp.int32),
        pltpu.VMEM((ECHUNK,), jnp.int32),
        pltpu.VMEM((N_GRAPHS * 9,), jnp.float32),
        pltpu.VMEM((ECHUNK,), jnp.float32),
        pltpu.VMEM((ECHUNK,), jnp.float32),
        pltpu.VMEM((ECHUNK,), jnp.float32),
        pltpu.VMEM((128,), jnp.int32),
        pltpu.VMEM((128, NODE_EMB), jnp.float32),
        pltpu.VMEM_SHARED((N_NODES,), jnp.float32),
        pltpu.VMEM_SHARED((N_NODES,), jnp.float32),
        pltpu.VMEM_SHARED((N_NODES,), jnp.float32),
        pltpu.SemaphoreType.DMA,
        pltpu.SemaphoreType.DMA,
        pltpu.SemaphoreType.DMA,
    ],
)

TBR = 64
TBE = TBR * 128
NROW = N_EDGES // 128
TGRID = (NROW + TBR - 1) // TBR


def _tc_body(vx_ref, vy_ref, vz_ref, w_ref, b_ref, dist_ref, emb_ref, rbf_t):
    x = vx_ref[...]
    y = vy_ref[...]
    z = vz_ref[...]
    d = jnp.sqrt(x * x + y * y + z * z + 1e-12)
    dist_ref[...] = d
    mu = lax.broadcasted_iota(jnp.int32, (N_RBF, 128), 0).astype(
        jnp.float32) * MU_STEP
    for r in range(TBR):
        db = jnp.broadcast_to(d[r:r + 1, :], (N_RBF, 128))
        delta = db - mu
        rbf_t[:, r * 128:(r + 1) * 128] = jnp.exp(-(delta * delta) * INV2S2)
    emb = lax.dot_general(rbf_t[...], w_ref[...], (((0,), (0,)), ((), ())),
                          preferred_element_type=jnp.float32)
    emb_ref[...] = emb + b_ref[...]


_tc_call = pl.pallas_call(
    _tc_body,
    grid=(TGRID,),
    in_specs=[
        pl.BlockSpec((TBR, 128), lambda i: (i, 0)),
        pl.BlockSpec((TBR, 128), lambda i: (i, 0)),
        pl.BlockSpec((TBR, 128), lambda i: (i, 0)),
        pl.BlockSpec((N_RBF, EDGE_EMB), lambda i: (0, 0)),
        pl.BlockSpec((1, EDGE_EMB), lambda i: (0, 0)),
    ],
    out_specs=[
        pl.BlockSpec((TBR, 128), lambda i: (i, 0)),
        pl.BlockSpec((TBE, EDGE_EMB), lambda i: (i, 0)),
    ],
    out_shape=[
        jax.ShapeDtypeStruct((NROW, 128), jnp.float32),
        jax.ShapeDtypeStruct((N_EDGES, EDGE_EMB), jnp.float32),
    ],
    scratch_shapes=[pltpu.VMEM((N_RBF, TBE), jnp.float32)],
)


def kernel(cart, lats, senders, receivers, to_jimage, edge_graph_i, species,
           W_proj, b_proj, species_table):
    lats_flat = lax.reduce_precision(lats, exponent_bits=8,
                                     mantissa_bits=7).reshape(-1)
    cartx, carty, cartz = cart[:, 0], cart[:, 1], cart[:, 2]
    vx, vy, vz, node_emb = _sc_call(cartx, carty, cartz, senders, receivers,
                                    to_jimage[:, 0], to_jimage[:, 1],
                                    to_jimage[:, 2], edge_graph_i,
                                    lats_flat, species, species_table)
    dist2d, edge_emb = _tc_call(vx.reshape(NROW, 128), vy.reshape(NROW, 128),
                                vz.reshape(NROW, 128), W_proj,
                                b_proj.reshape(1, EDGE_EMB))
    vecs = jnp.stack([vx, vy, vz], axis=1)
    return node_emb, edge_emb, vecs, dist2d.reshape(N_EDGES)


# speedup vs baseline: 5.2849x; 1.0062x over previous
import jax
import jax.numpy as jnp
from jax import lax
from jax.experimental import pallas as pl
from jax.experimental.pallas import tpu as pltpu
from jax.experimental.pallas import tpu_sc as plsc

N_NODES = 50000
N_EDGES = 800000
N_GRAPHS = 128
NODE_EMB = 128
N_RBF = 32
EDGE_EMB = 32
CUTOFF = 6.0
SIGMA = CUTOFF / N_RBF
INV2S2 = 1.0 / (2.0 * SIGMA * SIGMA)
MU_STEP = CUTOFF / (N_RBF - 1)

NC, NS = 2, 16
NW = NC * NS
EPW = N_EDGES // NW
ECHUNK = 1000
NECHUNK = EPW // ECHUNK
NODE_WORKERS = 25
NPW = N_NODES // NODE_WORKERS


def _sc_body(cartx, carty, cartz, senders, receivers, jx, jy, jz, gidx,
             lats_flat, species, table,
             vx_out, vy_out, vz_out, nemb_out,
             sidx, ridx, sxb, syb, szb, rxb, ryb, rzb,
             jxb, jyb, jzb, gb, latb, vxb, vyb, vzb, spid, nrows,
             cxsh, cysh, czsh, sem_in, sem_g, sem_n):
    sid = lax.axis_index("s")
    wid = sid * NC + lax.axis_index("c")

    pltpu.sync_copy(lats_flat, latb)

    @pl.when(sid == 0)
    def _():
        pltpu.sync_copy(cartx, cxsh)
        pltpu.sync_copy(carty, cysh)
        pltpu.sync_copy(cartz, czsh)
    plsc.subcore_barrier()

    iot = lax.iota(jnp.int32, 16)

    def edge_group(o):
        sx = sxb[pl.ds(o, 16)]
        sy = syb[pl.ds(o, 16)]
        sz = szb[pl.ds(o, 16)]
        rx = rxb[pl.ds(o, 16)]
        ry = ryb[pl.ds(o, 16)]
        rz = rzb[pl.ds(o, 16)]
        g9 = gb[pl.ds(o, 16)] * 9
        ja = jxb[pl.ds(o, 16)].astype(jnp.float32)
        jb_ = jyb[pl.ds(o, 16)].astype(jnp.float32)
        jc = jzb[pl.ds(o, 16)].astype(jnp.float32)
        l00 = plsc.load_gather(latb, [g9])
        l01 = plsc.load_gather(latb, [g9 + 1])
        l02 = plsc.load_gather(latb, [g9 + 2])
        l10 = plsc.load_gather(latb, [g9 + 3])
        l11 = plsc.load_gather(latb, [g9 + 4])
        l12 = plsc.load_gather(latb, [g9 + 5])
        l20 = plsc.load_gather(latb, [g9 + 6])
        l21 = plsc.load_gather(latb, [g9 + 7])
        l22 = plsc.load_gather(latb, [g9 + 8])
        vxb[pl.ds(o, 16)] = rx + (l00 * ja + l10 * jb_ + l20 * jc) - sx
        vyb[pl.ds(o, 16)] = ry + (l01 * ja + l11 * jb_ + l21 * jc) - sy
        vzb[pl.ds(o, 16)] = rz + (l02 * ja + l12 * jb_ + l22 * jc) - sz

    def chunk_body(ci, carry):
        base = wid * EPW + ci * ECHUNK
        sl_e = pl.ds(base, ECHUNK)
        cps = [pltpu.async_copy(senders.at[sl_e], sidx, sem_in),
               pltpu.async_copy(receivers.at[sl_e], ridx, sem_in),
               pltpu.async_copy(gidx.at[sl_e], gb, sem_in),
               pltpu.async_copy(jx.at[sl_e], jxb, sem_in),
               pltpu.async_copy(jy.at[sl_e], jyb, sem_in),
               pltpu.async_copy(jz.at[sl_e], jzb, sem_in)]
        for cp in cps:
            cp.wait()
        gcps = []
        for j in range(8):
            n = 128 if j < 7 else ECHUNK - 7 * 128
            sl = pl.ds(j * 128, n)
            for tab, idx, dst in ((cxsh, sidx, sxb), (cysh, sidx, syb),
                                  (czsh, sidx, szb), (cxsh, ridx, rxb),
                                  (cysh, ridx, ryb), (czsh, ridx, rzb)):
                gcps.append(pltpu.async_copy(tab.at[idx.at[sl]], dst.at[sl],
                                             sem_g))
        for cp in gcps:
            cp.wait()

        def g_body(i, c):
            edge_group(i * 16)
            return c
        lax.fori_loop(0, ECHUNK // 16, g_body, 0)
        edge_group(ECHUNK - 16)
        ocps = [pltpu.async_copy(vxb, vx_out.at[sl_e], sem_in),
                pltpu.async_copy(vyb, vy_out.at[sl_e], sem_in),
                pltpu.async_copy(vzb, vz_out.at[sl_e], sem_in)]
        for cp in ocps:
            cp.wait()
        return carry

    lax.fori_loop(0, NECHUNK, chunk_body, 0)

    @pl.when(wid < NODE_WORKERS)
    def _():
        nb = wid * NPW
        for j in range(16):
            n = 128 if j < 15 else NPW - 15 * 128
            sl = pl.ds(0, n)
            pltpu.sync_copy(species.at[pl.ds(nb + j * 128, n)], spid.at[sl])
            pltpu.async_copy(table.at[spid.at[sl]], nrows.at[sl],
                             sem_n).wait()
            pltpu.sync_copy(nrows.at[sl],
                            nemb_out.at[pl.ds(nb + j * 128, n)])


_sc_call = pl.kernel(
    _sc_body,
    out_type=[
        jax.ShapeDtypeStruct((N_EDGES,), jnp.float32),
        jax.ShapeDtypeStruct((N_EDGES,), jnp.float32),
        jax.ShapeDtypeStruct((N_EDGES,), jnp.float32),
        jax.ShapeDtypeStruct((N_NODES, NODE_EMB), jnp.float32),
    ],
    mesh=plsc.VectorSubcoreMesh(core_axis_name="c", subcore_axis_name="s"),
    compiler_params=pltpu.CompilerParams(needs_layout_passes=False,
                                         use_tc_tiling_on_sc=False),
    scratch_types=[
        pltpu.VMEM((ECHUNK,), jnp.int32),
        pltpu.VMEM((ECHUNK,), jnp.int32),
        pltpu.VMEM((ECHUNK,), jnp.float32),
        pltpu.VMEM((ECHUNK,), jnp.float32),
        pltpu.VMEM((ECHUNK,), jnp.float32),
        pltpu.VMEM((ECHUNK,), jnp.float32),
        pltpu.VMEM((ECHUNK,), jnp.float32),
        pltpu.VMEM((ECHUNK,), jnp.float32),
        pltpu.VMEM((ECHUNK,), jnp.int32),
        pltpu.VMEM((ECHUNK,), jnp.int32),
        pltpu.VMEM((ECHUNK,), jnp.int32),
        pltpu.VMEM((ECHUNK,), jnp.int32),
        pltpu.VMEM((N_GRAPHS * 9,), jnp.float32),
        pltpu.VMEM((ECHUNK,), jnp.float32),
        pltpu.VMEM((ECHUNK,), jnp.float32),
        pltpu.VMEM((ECHUNK,), jnp.float32),
        pltpu.VMEM((128,), jnp.int32),
        pltpu.VMEM((128, NODE_EMB), jnp.float32),
        pltpu.VMEM_SHARED((N_NODES,), jnp.float32),
        pltpu.VMEM_SHARED((N_NODES,), jnp.float32),
        pltpu.VMEM_SHARED((N_NODES,), jnp.float32),
        pltpu.SemaphoreType.DMA,
        pltpu.SemaphoreType.DMA,
        pltpu.SemaphoreType.DMA,
    ],
)

TBR = 64
TBE = TBR * 128
NROW = N_EDGES // 128
TGRID = (NROW + TBR - 1) // TBR


def _tc_body(vx_ref, vy_ref, vz_ref, w_ref, b_ref, dist_ref, emb_ref):
    x = vx_ref[...]
    y = vy_ref[...]
    z = vz_ref[...]
    d = jnp.sqrt(x * x + y * y + z * z + 1e-12)
    dist_ref[...] = d
    mu = lax.broadcasted_iota(jnp.int32, (N_RBF, TBE), 0).astype(
        jnp.float32) * MU_STEP
    db = jnp.broadcast_to(d.reshape(1, TBE), (N_RBF, TBE))
    delta = db - mu
    rbf_t = jnp.exp(-(delta * delta) * INV2S2)
    emb = lax.dot_general(rbf_t, w_ref[...], (((0,), (0,)), ((), ())),
                          preferred_element_type=jnp.float32)
    emb_ref[...] = emb + b_ref[...]


_tc_call = pl.pallas_call(
    _tc_body,
    grid=(TGRID,),
    in_specs=[
        pl.BlockSpec((TBE,), lambda i: (i,)),
        pl.BlockSpec((TBE,), lambda i: (i,)),
        pl.BlockSpec((TBE,), lambda i: (i,)),
        pl.BlockSpec((N_RBF, EDGE_EMB), lambda i: (0, 0)),
        pl.BlockSpec((1, EDGE_EMB), lambda i: (0, 0)),
    ],
    out_specs=[
        pl.BlockSpec((TBE,), lambda i: (i,)),
        pl.BlockSpec((TBE, EDGE_EMB), lambda i: (i, 0)),
    ],
    out_shape=[
        jax.ShapeDtypeStruct((N_EDGES,), jnp.float32),
        jax.ShapeDtypeStruct((N_EDGES, EDGE_EMB), jnp.float32),
    ],
)


def kernel(cart, lats, senders, receivers, to_jimage, edge_graph_i, species,
           W_proj, b_proj, species_table):
    lats_flat = lax.reduce_precision(lats, exponent_bits=8,
                                     mantissa_bits=7).reshape(-1)
    cartx, carty, cartz = cart[:, 0], cart[:, 1], cart[:, 2]
    vx, vy, vz, node_emb = _sc_call(cartx, carty, cartz, senders, receivers,
                                    to_jimage[:, 0], to_jimage[:, 1],
                                    to_jimage[:, 2], edge_graph_i,
                                    lats_flat, species, species_table)
    dist, edge_emb = _tc_call(vx, vy, vz, W_proj, b_proj.reshape(1, EDGE_EMB))
    vecs = jnp.stack([vx, vy, vz], axis=1)
    return node_emb, edge_emb, vecs, dist


# speedup vs baseline: 9.5590x; 1.8087x over previous
import jax
import jax.numpy as jnp
from jax import lax
from jax.experimental import pallas as pl
from jax.experimental.pallas import tpu as pltpu
from jax.experimental.pallas import tpu_sc as plsc

N_NODES = 50000
N_EDGES = 800000
N_GRAPHS = 128
NODE_EMB = 128
N_RBF = 32
EDGE_EMB = 32
CUTOFF = 6.0
SIGMA = CUTOFF / N_RBF
INV2S2 = 1.0 / (2.0 * SIGMA * SIGMA)
MU_STEP = CUTOFF / (N_RBF - 1)

NC, NS = 2, 16
NW = NC * NS
EPW = N_EDGES // NW
ECHUNK = 1000
NECHUNK = EPW // ECHUNK
NODE_WORKERS = 25
NPW = N_NODES // NODE_WORKERS


def _sc_body(cartx, carty, cartz, senders, receivers, jx, jy, jz, gidx,
             lats_flat, species, table,
             vx_out, vy_out, vz_out, nemb_out,
             sidx, ridx, sxb, syb, szb, rxb, ryb, rzb,
             jxb, jyb, jzb, gb, latb, vxb, vyb, vzb, spid, nrows,
             cxsh, cysh, czsh, sem_in, sem_g, sem_n):
    sid = lax.axis_index("s")
    wid = sid * NC + lax.axis_index("c")

    pltpu.sync_copy(lats_flat, latb)

    @pl.when(sid == 0)
    def _():
        pltpu.sync_copy(cartx, cxsh)
        pltpu.sync_copy(carty, cysh)
        pltpu.sync_copy(cartz, czsh)
    plsc.subcore_barrier()

    iot = lax.iota(jnp.int32, 16)

    def edge_group(o):
        sx = sxb[pl.ds(o, 16)]
        sy = syb[pl.ds(o, 16)]
        sz = szb[pl.ds(o, 16)]
        rx = rxb[pl.ds(o, 16)]
        ry = ryb[pl.ds(o, 16)]
        rz = rzb[pl.ds(o, 16)]
        g9 = gb[pl.ds(o, 16)] * 9
        ja = jxb[pl.ds(o, 16)].astype(jnp.float32)
        jb_ = jyb[pl.ds(o, 16)].astype(jnp.float32)
        jc = jzb[pl.ds(o, 16)].astype(jnp.float32)
        l00 = plsc.load_gather(latb, [g9])
        l01 = plsc.load_gather(latb, [g9 + 1])
        l02 = plsc.load_gather(latb, [g9 + 2])
        l10 = plsc.load_gather(latb, [g9 + 3])
        l11 = plsc.load_gather(latb, [g9 + 4])
        l12 = plsc.load_gather(latb, [g9 + 5])
        l20 = plsc.load_gather(latb, [g9 + 6])
        l21 = plsc.load_gather(latb, [g9 + 7])
        l22 = plsc.load_gather(latb, [g9 + 8])
        vxb[pl.ds(o, 16)] = rx + (l00 * ja + l10 * jb_ + l20 * jc) - sx
        vyb[pl.ds(o, 16)] = ry + (l01 * ja + l11 * jb_ + l21 * jc) - sy
        vzb[pl.ds(o, 16)] = rz + (l02 * ja + l12 * jb_ + l22 * jc) - sz

    def chunk_body(ci, carry):
        base = wid * EPW + ci * ECHUNK
        sl_e = pl.ds(base, ECHUNK)
        cps = [pltpu.async_copy(senders.at[sl_e], sidx, sem_in),
               pltpu.async_copy(receivers.at[sl_e], ridx, sem_in),
               pltpu.async_copy(gidx.at[sl_e], gb, sem_in),
               pltpu.async_copy(jx.at[sl_e], jxb, sem_in),
               pltpu.async_copy(jy.at[sl_e], jyb, sem_in),
               pltpu.async_copy(jz.at[sl_e], jzb, sem_in)]
        for cp in cps:
            cp.wait()
        gcps = []
        for j in range(8):
            n = 128 if j < 7 else ECHUNK - 7 * 128
            sl = pl.ds(j * 128, n)
            for tab, idx, dst in ((cxsh, sidx, sxb), (cysh, sidx, syb),
                                  (czsh, sidx, szb), (cxsh, ridx, rxb),
                                  (cysh, ridx, ryb), (czsh, ridx, rzb)):
                gcps.append(pltpu.async_copy(tab.at[idx.at[sl]], dst.at[sl],
                                             sem_g))
        for cp in gcps:
            cp.wait()

        def g_body(i, c):
            edge_group(i * 16)
            return c
        lax.fori_loop(0, ECHUNK // 16, g_body, 0)
        edge_group(ECHUNK - 16)
        ocps = [pltpu.async_copy(vxb, vx_out.at[sl_e], sem_in),
                pltpu.async_copy(vyb, vy_out.at[sl_e], sem_in),
                pltpu.async_copy(vzb, vz_out.at[sl_e], sem_in)]
        for cp in ocps:
            cp.wait()
        return carry

    lax.fori_loop(0, NECHUNK, chunk_body, 0)

    @pl.when(wid < NODE_WORKERS)
    def _():
        nb = wid * NPW
        for j in range(16):
            n = 128 if j < 15 else NPW - 15 * 128
            sl = pl.ds(0, n)
            pltpu.sync_copy(species.at[pl.ds(nb + j * 128, n)], spid.at[sl])
            pltpu.async_copy(table.at[spid.at[sl]], nrows.at[sl],
                             sem_n).wait()
            pltpu.sync_copy(nrows.at[sl],
                            nemb_out.at[pl.ds(nb + j * 128, n)])


_sc_call = pl.kernel(
    _sc_body,
    out_type=[
        jax.ShapeDtypeStruct((N_EDGES,), jnp.float32),
        jax.ShapeDtypeStruct((N_EDGES,), jnp.float32),
        jax.ShapeDtypeStruct((N_EDGES,), jnp.float32),
        jax.ShapeDtypeStruct((N_NODES, NODE_EMB), jnp.float32),
    ],
    mesh=plsc.VectorSubcoreMesh(core_axis_name="c", subcore_axis_name="s"),
    compiler_params=pltpu.CompilerParams(needs_layout_passes=False,
                                         use_tc_tiling_on_sc=False),
    scratch_types=[
        pltpu.VMEM((ECHUNK,), jnp.int32),
        pltpu.VMEM((ECHUNK,), jnp.int32),
        pltpu.VMEM((ECHUNK,), jnp.float32),
        pltpu.VMEM((ECHUNK,), jnp.float32),
        pltpu.VMEM((ECHUNK,), jnp.float32),
        pltpu.VMEM((ECHUNK,), jnp.float32),
        pltpu.VMEM((ECHUNK,), jnp.float32),
        pltpu.VMEM((ECHUNK,), jnp.float32),
        pltpu.VMEM((ECHUNK,), jnp.int32),
        pltpu.VMEM((ECHUNK,), jnp.int32),
        pltpu.VMEM((ECHUNK,), jnp.int32),
        pltpu.VMEM((ECHUNK,), jnp.int32),
        pltpu.VMEM((N_GRAPHS * 9,), jnp.float32),
        pltpu.VMEM((ECHUNK,), jnp.float32),
        pltpu.VMEM((ECHUNK,), jnp.float32),
        pltpu.VMEM((ECHUNK,), jnp.float32),
        pltpu.VMEM((128,), jnp.int32),
        pltpu.VMEM((128, NODE_EMB), jnp.float32),
        pltpu.VMEM_SHARED((N_NODES,), jnp.float32),
        pltpu.VMEM_SHARED((N_NODES,), jnp.float32),
        pltpu.VMEM_SHARED((N_NODES,), jnp.float32),
        pltpu.SemaphoreType.DMA,
        pltpu.SemaphoreType.DMA,
        pltpu.SemaphoreType.DMA,
    ],
)

TBR = 64
TBE = TBR * 128
NROW = N_EDGES // 128
TGRID = (NROW + TBR - 1) // TBR


def _tc_body(vx_ref, vy_ref, vz_ref, w_ref, b_ref, dist_ref, emb_ref):
    x = vx_ref[...]
    y = vy_ref[...]
    z = vz_ref[...]
    d = jnp.sqrt(x * x + y * y + z * z + 1e-12)
    dist_ref[...] = d
    mu = lax.broadcasted_iota(jnp.int32, (N_RBF, TBE), 0).astype(
        jnp.float32) * MU_STEP
    db = jnp.broadcast_to(d.reshape(1, TBE), (N_RBF, TBE))
    delta = db - mu
    rbf_t = jnp.exp(-(delta * delta) * INV2S2)
    embT = lax.dot_general(w_ref[...], rbf_t, (((0,), (0,)), ((), ())),
                           preferred_element_type=jnp.float32)
    emb_ref[...] = embT + b_ref[...]


_tc_call = pl.pallas_call(
    _tc_body,
    grid=(TGRID,),
    in_specs=[
        pl.BlockSpec((TBE,), lambda i: (i,)),
        pl.BlockSpec((TBE,), lambda i: (i,)),
        pl.BlockSpec((TBE,), lambda i: (i,)),
        pl.BlockSpec((N_RBF, EDGE_EMB), lambda i: (0, 0)),
        pl.BlockSpec((EDGE_EMB, 1), lambda i: (0, 0)),
    ],
    out_specs=[
        pl.BlockSpec((TBE,), lambda i: (i,)),
        pl.BlockSpec((EDGE_EMB, TBE), lambda i: (0, i)),
    ],
    out_shape=[
        jax.ShapeDtypeStruct((N_EDGES,), jnp.float32),
        jax.ShapeDtypeStruct((EDGE_EMB, N_EDGES), jnp.float32),
    ],
)


def kernel(cart, lats, senders, receivers, to_jimage, edge_graph_i, species,
           W_proj, b_proj, species_table):
    lats_flat = lax.reduce_precision(lats, exponent_bits=8,
                                     mantissa_bits=7).reshape(-1)
    cartx, carty, cartz = cart[:, 0], cart[:, 1], cart[:, 2]
    vx, vy, vz, node_emb = _sc_call(cartx, carty, cartz, senders, receivers,
                                    to_jimage[:, 0], to_jimage[:, 1],
                                    to_jimage[:, 2], edge_graph_i,
                                    lats_flat, species, species_table)
    dist, embT = _tc_call(vx, vy, vz, W_proj, b_proj.reshape(EDGE_EMB, 1))
    vecs = jnp.stack([vx, vy, vz], axis=1)
    return node_emb, embT.T, vecs, dist


# speedup vs baseline: 9.5725x; 1.0014x over previous
import jax
import jax.numpy as jnp
from jax import lax
from jax.experimental import pallas as pl
from jax.experimental.pallas import tpu as pltpu
from jax.experimental.pallas import tpu_sc as plsc

N_NODES = 50000
N_EDGES = 800000
N_GRAPHS = 128
NODE_EMB = 128
N_RBF = 32
EDGE_EMB = 32
CUTOFF = 6.0
SIGMA = CUTOFF / N_RBF
INV2S2 = 1.0 / (2.0 * SIGMA * SIGMA)
MU_STEP = CUTOFF / (N_RBF - 1)

NC, NS = 2, 16
NW = NC * NS
EPW = N_EDGES // NW
ECHUNK = 1000
NECHUNK = EPW // ECHUNK
NODE_WORKERS = 25
NPW = N_NODES // NODE_WORKERS


def _sc_body(cartx, carty, cartz, senders, receivers, jx, jy, jz, gidx,
             lats_flat, species, table,
             vx_out, vy_out, vz_out, nemb_out,
             sidx, ridx, sxb, syb, szb, rxb, ryb, rzb,
             jxb, jyb, jzb, gb, latb, vxb, vyb, vzb, spid, nrows,
             cxsh, cysh, czsh, sem_in, sem_g, sem_n):
    sid = lax.axis_index("s")
    wid = sid * NC + lax.axis_index("c")

    pltpu.sync_copy(lats_flat, latb)

    @pl.when(sid == 0)
    def _():
        pltpu.sync_copy(cartx, cxsh)
        pltpu.sync_copy(carty, cysh)
        pltpu.sync_copy(cartz, czsh)
    plsc.subcore_barrier()

    iot = lax.iota(jnp.int32, 16)

    def edge_group(o):
        sx = sxb[pl.ds(o, 16)]
        sy = syb[pl.ds(o, 16)]
        sz = szb[pl.ds(o, 16)]
        rx = rxb[pl.ds(o, 16)]
        ry = ryb[pl.ds(o, 16)]
        rz = rzb[pl.ds(o, 16)]
        g9 = gb[pl.ds(o, 16)] * 9
        ja = jxb[pl.ds(o, 16)].astype(jnp.float32)
        jb_ = jyb[pl.ds(o, 16)].astype(jnp.float32)
        jc = jzb[pl.ds(o, 16)].astype(jnp.float32)
        l00 = plsc.load_gather(latb, [g9])
        l01 = plsc.load_gather(latb, [g9 + 1])
        l02 = plsc.load_gather(latb, [g9 + 2])
        l10 = plsc.load_gather(latb, [g9 + 3])
        l11 = plsc.load_gather(latb, [g9 + 4])
        l12 = plsc.load_gather(latb, [g9 + 5])
        l20 = plsc.load_gather(latb, [g9 + 6])
        l21 = plsc.load_gather(latb, [g9 + 7])
        l22 = plsc.load_gather(latb, [g9 + 8])
        vxb[pl.ds(o, 16)] = rx + (l00 * ja + l10 * jb_ + l20 * jc) - sx
        vyb[pl.ds(o, 16)] = ry + (l01 * ja + l11 * jb_ + l21 * jc) - sy
        vzb[pl.ds(o, 16)] = rz + (l02 * ja + l12 * jb_ + l22 * jc) - sz

    def chunk_body(ci, carry):
        base = wid * EPW + ci * ECHUNK
        sl_e = pl.ds(base, ECHUNK)
        cps = [pltpu.async_copy(senders.at[sl_e], sidx, sem_in),
               pltpu.async_copy(receivers.at[sl_e], ridx, sem_in),
               pltpu.async_copy(gidx.at[sl_e], gb, sem_in),
               pltpu.async_copy(jx.at[sl_e], jxb, sem_in),
               pltpu.async_copy(jy.at[sl_e], jyb, sem_in),
               pltpu.async_copy(jz.at[sl_e], jzb, sem_in)]
        for cp in cps:
            cp.wait()
        gcps = []
        for j in range(8):
            n = 128 if j < 7 else ECHUNK - 7 * 128
            sl = pl.ds(j * 128, n)
            for tab, idx, dst in ((cxsh, sidx, sxb), (cysh, sidx, syb),
                                  (czsh, sidx, szb), (cxsh, ridx, rxb),
                                  (cysh, ridx, ryb), (czsh, ridx, rzb)):
                gcps.append(pltpu.async_copy(tab.at[idx.at[sl]], dst.at[sl],
                                             sem_g))
        for cp in gcps:
            cp.wait()

        def g_body(i, c):
            edge_group(i * 16)
            return c
        lax.fori_loop(0, ECHUNK // 16, g_body, 0)
        edge_group(ECHUNK - 16)
        ocps = [pltpu.async_copy(vxb, vx_out.at[sl_e], sem_in),
                pltpu.async_copy(vyb, vy_out.at[sl_e], sem_in),
                pltpu.async_copy(vzb, vz_out.at[sl_e], sem_in)]
        for cp in ocps:
            cp.wait()
        return carry

    lax.fori_loop(0, NECHUNK, chunk_body, 0)

    @pl.when(wid < NODE_WORKERS)
    def _():
        nb = wid * NPW
        pltpu.sync_copy(species.at[pl.ds(nb, NPW)], spid)
        for r in range(4):
            cnt = 512 if r < 3 else NPW - 3 * 512
            nfull, tail = cnt // 128, cnt % 128
            gcs = []
            for q in range(nfull):
                gcs.append(pltpu.async_copy(
                    table.at[spid.at[pl.ds(r * 512 + q * 128, 128)]],
                    nrows.at[pl.ds(q * 128, 128)], sem_n))
            if tail:
                gcs.append(pltpu.async_copy(
                    table.at[spid.at[pl.ds(r * 512 + nfull * 128, tail)]],
                    nrows.at[pl.ds(nfull * 128, tail)], sem_n))
            for cp in gcs:
                cp.wait()
            pltpu.sync_copy(nrows.at[pl.ds(0, cnt)],
                            nemb_out.at[pl.ds(nb + r * 512, cnt)])


_sc_call = pl.kernel(
    _sc_body,
    out_type=[
        jax.ShapeDtypeStruct((N_EDGES,), jnp.float32),
        jax.ShapeDtypeStruct((N_EDGES,), jnp.float32),
        jax.ShapeDtypeStruct((N_EDGES,), jnp.float32),
        jax.ShapeDtypeStruct((N_NODES, NODE_EMB), jnp.float32),
    ],
    mesh=plsc.VectorSubcoreMesh(core_axis_name="c", subcore_axis_name="s"),
    compiler_params=pltpu.CompilerParams(needs_layout_passes=False,
                                         use_tc_tiling_on_sc=False),
    scratch_types=[
        pltpu.VMEM((ECHUNK,), jnp.int32),
        pltpu.VMEM((ECHUNK,), jnp.int32),
        pltpu.VMEM((ECHUNK,), jnp.float32),
        pltpu.VMEM((ECHUNK,), jnp.float32),
        pltpu.VMEM((ECHUNK,), jnp.float32),
        pltpu.VMEM((ECHUNK,), jnp.float32),
        pltpu.VMEM((ECHUNK,), jnp.float32),
        pltpu.VMEM((ECHUNK,), jnp.float32),
        pltpu.VMEM((ECHUNK,), jnp.int32),
        pltpu.VMEM((ECHUNK,), jnp.int32),
        pltpu.VMEM((ECHUNK,), jnp.int32),
        pltpu.VMEM((ECHUNK,), jnp.int32),
        pltpu.VMEM((N_GRAPHS * 9,), jnp.float32),
        pltpu.VMEM((ECHUNK,), jnp.float32),
        pltpu.VMEM((ECHUNK,), jnp.float32),
        pltpu.VMEM((ECHUNK,), jnp.float32),
        pltpu.VMEM((NPW,), jnp.int32),
        pltpu.VMEM((512, NODE_EMB), jnp.float32),
        pltpu.VMEM_SHARED((N_NODES,), jnp.float32),
        pltpu.VMEM_SHARED((N_NODES,), jnp.float32),
        pltpu.VMEM_SHARED((N_NODES,), jnp.float32),
        pltpu.SemaphoreType.DMA,
        pltpu.SemaphoreType.DMA,
        pltpu.SemaphoreType.DMA,
    ],
)

TBR = 64
TBE = TBR * 128
NROW = N_EDGES // 128
TGRID = (NROW + TBR - 1) // TBR


def _tc_body(vx_ref, vy_ref, vz_ref, w_ref, b_ref, dist_ref, emb_ref):
    x = vx_ref[...]
    y = vy_ref[...]
    z = vz_ref[...]
    d = jnp.sqrt(x * x + y * y + z * z + 1e-12)
    dist_ref[...] = d
    mu = lax.broadcasted_iota(jnp.int32, (N_RBF, TBE), 0).astype(
        jnp.float32) * MU_STEP
    db = jnp.broadcast_to(d.reshape(1, TBE), (N_RBF, TBE))
    delta = db - mu
    rbf_t = jnp.exp(-(delta * delta) * INV2S2)
    embT = lax.dot_general(w_ref[...], rbf_t, (((0,), (0,)), ((), ())),
                           preferred_element_type=jnp.float32)
    emb_ref[...] = embT + b_ref[...]


_tc_call = pl.pallas_call(
    _tc_body,
    grid=(TGRID,),
    in_specs=[
        pl.BlockSpec((TBE,), lambda i: (i,)),
        pl.BlockSpec((TBE,), lambda i: (i,)),
        pl.BlockSpec((TBE,), lambda i: (i,)),
        pl.BlockSpec((N_RBF, EDGE_EMB), lambda i: (0, 0)),
        pl.BlockSpec((EDGE_EMB, 1), lambda i: (0, 0)),
    ],
    out_specs=[
        pl.BlockSpec((TBE,), lambda i: (i,)),
        pl.BlockSpec((EDGE_EMB, TBE), lambda i: (0, i)),
    ],
    out_shape=[
        jax.ShapeDtypeStruct((N_EDGES,), jnp.float32),
        jax.ShapeDtypeStruct((EDGE_EMB, N_EDGES), jnp.float32),
    ],
)


def kernel(cart, lats, senders, receivers, to_jimage, edge_graph_i, species,
           W_proj, b_proj, species_table):
    lats_flat = lax.reduce_precision(lats, exponent_bits=8,
                                     mantissa_bits=7).reshape(-1)
    cartx, carty, cartz = cart[:, 0], cart[:, 1], cart[:, 2]
    vx, vy, vz, node_emb = _sc_call(cartx, carty, cartz, senders, receivers,
                                    to_jimage[:, 0], to_jimage[:, 1],
                                    to_jimage[:, 2], edge_graph_i,
                                    lats_flat, species, species_table)
    dist, embT = _tc_call(vx, vy, vz, W_proj, b_proj.reshape(EDGE_EMB, 1))
    vecs = jnp.stack([vx, vy, vz], axis=1)
    return node_emb, embT.T, vecs, dist


# speedup vs baseline: 9.6343x; 1.0065x over previous
import jax
import jax.numpy as jnp
from jax import lax
from jax.experimental import pallas as pl
from jax.experimental.pallas import tpu as pltpu
from jax.experimental.pallas import tpu_sc as plsc

N_NODES = 50000
N_EDGES = 800000
N_GRAPHS = 128
NODE_EMB = 128
N_RBF = 32
EDGE_EMB = 32
CUTOFF = 6.0
SIGMA = CUTOFF / N_RBF
INV2S2 = 1.0 / (2.0 * SIGMA * SIGMA)
MU_STEP = CUTOFF / (N_RBF - 1)

NC, NS = 2, 16
NW = NC * NS
EPW = N_EDGES // NW
ECHUNK = 1000
NECHUNK = EPW // ECHUNK
NODE_WORKERS = 25
NPW = N_NODES // NODE_WORKERS


def _sc_body(cartx, carty, cartz, senders, receivers, jx, jy, jz, gidx,
             lats_flat, species, table,
             vx_out, vy_out, vz_out, nemb_out,
             sidx, ridx, sxb, syb, szb, rxb, ryb, rzb,
             jxb, jyb, jzb, gb, latb, vxb, vyb, vzb, spid, nrows,
             cxsh, cysh, czsh, sem_in, sem_g, sem_g2, sem_n):
    sid = lax.axis_index("s")
    wid = sid * NC + lax.axis_index("c")

    pltpu.sync_copy(lats_flat, latb)

    @pl.when(sid == 0)
    def _():
        pltpu.sync_copy(cartx, cxsh)
        pltpu.sync_copy(carty, cysh)
        pltpu.sync_copy(cartz, czsh)
    plsc.subcore_barrier()

    iot = lax.iota(jnp.int32, 16)

    def edge_group(o):
        sx = sxb[pl.ds(o, 16)]
        sy = syb[pl.ds(o, 16)]
        sz = szb[pl.ds(o, 16)]
        rx = rxb[pl.ds(o, 16)]
        ry = ryb[pl.ds(o, 16)]
        rz = rzb[pl.ds(o, 16)]
        g9 = gb[pl.ds(o, 16)] * 9
        ja = jxb[pl.ds(o, 16)].astype(jnp.float32)
        jb_ = jyb[pl.ds(o, 16)].astype(jnp.float32)
        jc = jzb[pl.ds(o, 16)].astype(jnp.float32)
        l00 = plsc.load_gather(latb, [g9])
        l01 = plsc.load_gather(latb, [g9 + 1])
        l02 = plsc.load_gather(latb, [g9 + 2])
        l10 = plsc.load_gather(latb, [g9 + 3])
        l11 = plsc.load_gather(latb, [g9 + 4])
        l12 = plsc.load_gather(latb, [g9 + 5])
        l20 = plsc.load_gather(latb, [g9 + 6])
        l21 = plsc.load_gather(latb, [g9 + 7])
        l22 = plsc.load_gather(latb, [g9 + 8])
        vxb[pl.ds(o, 16)] = rx + (l00 * ja + l10 * jb_ + l20 * jc) - sx
        vyb[pl.ds(o, 16)] = ry + (l01 * ja + l11 * jb_ + l21 * jc) - sy
        vzb[pl.ds(o, 16)] = rz + (l02 * ja + l12 * jb_ + l22 * jc) - sz

    def chunk_body(ci, carry):
        base = wid * EPW + ci * ECHUNK
        sl_e = pl.ds(base, ECHUNK)
        cps = [pltpu.async_copy(senders.at[sl_e], sidx, sem_in),
               pltpu.async_copy(receivers.at[sl_e], ridx, sem_in),
               pltpu.async_copy(gidx.at[sl_e], gb, sem_in),
               pltpu.async_copy(jx.at[sl_e], jxb, sem_in),
               pltpu.async_copy(jy.at[sl_e], jyb, sem_in),
               pltpu.async_copy(jz.at[sl_e], jzb, sem_in)]
        for cp in cps:
            cp.wait()
        tables = ((cxsh, sidx, sxb), (cysh, sidx, syb), (czsh, sidx, szb),
                  (cxsh, ridx, rxb), (cysh, ridx, ryb), (czsh, ridx, rzb))
        gca, gcb = [], []
        for j in range(4):
            sl = pl.ds(j * 128, 128)
            for tab, idx, dst in tables:
                gca.append(pltpu.async_copy(tab.at[idx.at[sl]], dst.at[sl],
                                            sem_g))
        for j in range(4):
            n = 128 if j < 3 else ECHUNK - 512 - 3 * 128
            sl = pl.ds(512 + j * 128, n)
            for tab, idx, dst in tables:
                gcb.append(pltpu.async_copy(tab.at[idx.at[sl]], dst.at[sl],
                                            sem_g2))

        def g_body(i, c):
            edge_group(i * 16)
            return c
        for cp in gca:
            cp.wait()
        lax.fori_loop(0, 32, g_body, 0)
        for cp in gcb:
            cp.wait()
        lax.fori_loop(32, (ECHUNK // 16) - 1, g_body, 0)
        edge_group(ECHUNK - 32)
        edge_group(ECHUNK - 16)
        ocps = [pltpu.async_copy(vxb, vx_out.at[sl_e], sem_in),
                pltpu.async_copy(vyb, vy_out.at[sl_e], sem_in),
                pltpu.async_copy(vzb, vz_out.at[sl_e], sem_in)]
        for cp in ocps:
            cp.wait()
        return carry

    lax.fori_loop(0, NECHUNK, chunk_body, 0)

    @pl.when(wid < NODE_WORKERS)
    def _():
        nb = wid * NPW
        pltpu.sync_copy(species.at[pl.ds(nb, NPW)], spid)
        for r in range(4):
            cnt = 512 if r < 3 else NPW - 3 * 512
            nfull, tail = cnt // 128, cnt % 128
            gcs = []
            for q in range(nfull):
                gcs.append(pltpu.async_copy(
                    table.at[spid.at[pl.ds(r * 512 + q * 128, 128)]],
                    nrows.at[pl.ds(q * 128, 128)], sem_n))
            if tail:
                gcs.append(pltpu.async_copy(
                    table.at[spid.at[pl.ds(r * 512 + nfull * 128, tail)]],
                    nrows.at[pl.ds(nfull * 128, tail)], sem_n))
            for cp in gcs:
                cp.wait()
            pltpu.sync_copy(nrows.at[pl.ds(0, cnt)],
                            nemb_out.at[pl.ds(nb + r * 512, cnt)])


_sc_call = pl.kernel(
    _sc_body,
    out_type=[
        jax.ShapeDtypeStruct((N_EDGES,), jnp.float32),
        jax.ShapeDtypeStruct((N_EDGES,), jnp.float32),
        jax.ShapeDtypeStruct((N_EDGES,), jnp.float32),
        jax.ShapeDtypeStruct((N_NODES, NODE_EMB), jnp.float32),
    ],
    mesh=plsc.VectorSubcoreMesh(core_axis_name="c", subcore_axis_name="s"),
    compiler_params=pltpu.CompilerParams(needs_layout_passes=False,
                                         use_tc_tiling_on_sc=False),
    scratch_types=[
        pltpu.VMEM((ECHUNK,), jnp.int32),
        pltpu.VMEM((ECHUNK,), jnp.int32),
        pltpu.VMEM((ECHUNK,), jnp.float32),
        pltpu.VMEM((ECHUNK,), jnp.float32),
        pltpu.VMEM((ECHUNK,), jnp.float32),
        pltpu.VMEM((ECHUNK,), jnp.float32),
        pltpu.VMEM((ECHUNK,), jnp.float32),
        pltpu.VMEM((ECHUNK,), jnp.float32),
        pltpu.VMEM((ECHUNK,), jnp.int32),
        pltpu.VMEM((ECHUNK,), jnp.int32),
        pltpu.VMEM((ECHUNK,), jnp.int32),
        pltpu.VMEM((ECHUNK,), jnp.int32),
        pltpu.VMEM((N_GRAPHS * 9,), jnp.float32),
        pltpu.VMEM((ECHUNK,), jnp.float32),
        pltpu.VMEM((ECHUNK,), jnp.float32),
        pltpu.VMEM((ECHUNK,), jnp.float32),
        pltpu.VMEM((NPW,), jnp.int32),
        pltpu.VMEM((512, NODE_EMB), jnp.float32),
        pltpu.VMEM_SHARED((N_NODES,), jnp.float32),
        pltpu.VMEM_SHARED((N_NODES,), jnp.float32),
        pltpu.VMEM_SHARED((N_NODES,), jnp.float32),
        pltpu.SemaphoreType.DMA,
        pltpu.SemaphoreType.DMA,
        pltpu.SemaphoreType.DMA,
        pltpu.SemaphoreType.DMA,
    ],
)

TBR = 64
TBE = TBR * 128
NROW = N_EDGES // 128
TGRID = (NROW + TBR - 1) // TBR


def _tc_body(vx_ref, vy_ref, vz_ref, w_ref, b_ref, dist_ref, emb_ref):
    x = vx_ref[...]
    y = vy_ref[...]
    z = vz_ref[...]
    d = jnp.sqrt(x * x + y * y + z * z + 1e-12)
    dist_ref[...] = d
    mu = lax.broadcasted_iota(jnp.int32, (N_RBF, TBE), 0).astype(
        jnp.float32) * MU_STEP
    db = jnp.broadcast_to(d.reshape(1, TBE), (N_RBF, TBE))
    delta = db - mu
    rbf_t = jnp.exp(-(delta * delta) * INV2S2)
    embT = lax.dot_general(w_ref[...], rbf_t, (((0,), (0,)), ((), ())),
                           preferred_element_type=jnp.float32)
    emb_ref[...] = embT + b_ref[...]


_tc_call = pl.pallas_call(
    _tc_body,
    grid=(TGRID,),
    in_specs=[
        pl.BlockSpec((TBE,), lambda i: (i,)),
        pl.BlockSpec((TBE,), lambda i: (i,)),
        pl.BlockSpec((TBE,), lambda i: (i,)),
        pl.BlockSpec((N_RBF, EDGE_EMB), lambda i: (0, 0)),
        pl.BlockSpec((EDGE_EMB, 1), lambda i: (0, 0)),
    ],
    out_specs=[
        pl.BlockSpec((TBE,), lambda i: (i,)),
        pl.BlockSpec((EDGE_EMB, TBE), lambda i: (0, i)),
    ],
    out_shape=[
        jax.ShapeDtypeStruct((N_EDGES,), jnp.float32),
        jax.ShapeDtypeStruct((EDGE_EMB, N_EDGES), jnp.float32),
    ],
)


def kernel(cart, lats, senders, receivers, to_jimage, edge_graph_i, species,
           W_proj, b_proj, species_table):
    lats_flat = lax.reduce_precision(lats, exponent_bits=8,
                                     mantissa_bits=7).reshape(-1)
    cartx, carty, cartz = cart[:, 0], cart[:, 1], cart[:, 2]
    vx, vy, vz, node_emb = _sc_call(cartx, carty, cartz, senders, receivers,
                                    to_jimage[:, 0], to_jimage[:, 1],
                                    to_jimage[:, 2], edge_graph_i,
                                    lats_flat, species, species_table)
    dist, embT = _tc_call(vx, vy, vz, W_proj, b_proj.reshape(EDGE_EMB, 1))
    vecs = jnp.stack([vx, vy, vz], axis=1)
    return node_emb, embT.T, vecs, dist
